# baseline (speedup 1.0000x reference)
import functools

import jax
import jax.numpy as jnp
from jax import lax
from jax.experimental import pallas as pl
from jax.experimental.pallas import tpu as pltpu
from jax.experimental.pallas import tpu_sc as plsc

N = 10000
D = 128
NUM_LAYERS = 4
NC = 2
NS = 16
NW = NC * NS
BLK = 128
ACC_ROWS_PER_TILE = 632
NPAD = NS * ACC_ROWS_PER_TILE
OUT_ROWS = ACC_ROWS_PER_TILE
DW = 16
R = 1000
GRID = N // R


def _make_propagate(kb):
    mesh = plsc.VectorSubcoreMesh(core_axis_name="c", subcore_axis_name="s")
    kbp = kb + 2

    @functools.partial(
        pl.kernel,
        out_type=jax.ShapeDtypeStruct((NC, NPAD, D), jnp.float32),
        mesh=mesh,
        scratch_types=[
            pltpu.VMEM_SHARED((NPAD, D), jnp.float32),
            pltpu.VMEM((BLK, D), jnp.float32),
            pltpu.VMEM((kbp * BLK,), jnp.int32),
            pltpu.VMEM((kb, BLK), jnp.int32),
            pltpu.SemaphoreType.DMA,
            pltpu.SemaphoreType.DMA,
        ],
    )
    def propagate(y_hbm, rowp_hbm, colp_hbm, out_hbm, acc, buf0,
                  idx_r, idx_c, sem0, sem1):
        c = lax.axis_index("c")
        s = lax.axis_index("s")
        t = s * NC + c
        tb = t * kbp * BLK

        pltpu.async_copy(rowp_hbm.at[pl.ds(tb, kbp * BLK)], idx_r, sem0)
        pltpu.async_copy(colp_hbm.at[t], idx_c, sem1)

        zeros16 = jnp.zeros((16,), jnp.float32)

        def _zrow(i, _):
            def _zcol(j, _):
                buf0[i, pl.ds(j * 16, 16)] = zeros16
                return 0
            return lax.fori_loop(0, D // 16, _zcol, 0)

        lax.fori_loop(0, BLK, _zrow, 0)

        zb = s * ACC_ROWS_PER_TILE
        for k in range(ACC_ROWS_PER_TILE // BLK):
            pltpu.sync_copy(buf0, acc.at[pl.ds(zb + k * BLK, BLK)])
        rem = ACC_ROWS_PER_TILE % BLK
        pltpu.sync_copy(buf0.at[pl.ds(0, rem)],
                        acc.at[pl.ds(zb + (ACC_ROWS_PER_TILE // BLK) * BLK, rem)])

        pltpu.make_async_copy(rowp_hbm.at[pl.ds(0, kbp * BLK)], idx_r, sem0).wait()
        pltpu.make_async_copy(colp_hbm.at[t], idx_c, sem1).wait()
        plsc.subcore_barrier()

        def _gidx(j):
            return idx_r.at[pl.ds(j * BLK, BLK)]

        def _edge_block(j, _):
            pltpu.async_copy(y_hbm.at[_gidx(j)], buf0, sem0).wait()
            pltpu.sync_copy(buf0, acc.at[idx_c.at[j]], add=True)
            return 0

        lax.fori_loop(0, kb, _edge_block, 0)
        plsc.subcore_barrier()

        ob = s * OUT_ROWS
        pltpu.sync_copy(acc.at[pl.ds(ob, OUT_ROWS)], out_hbm.at[c, pl.ds(ob, OUT_ROWS)])

    return propagate


def _make_degree(kb):
    mesh = plsc.VectorSubcoreMesh(core_axis_name="c", subcore_axis_name="s")
    kbp = kb + 2

    @functools.partial(
        pl.kernel,
        out_type=jax.ShapeDtypeStruct((NC, NPAD, DW), jnp.float32),
        mesh=mesh,
        scratch_types=[
            pltpu.VMEM_SHARED((NPAD, DW), jnp.float32),
            pltpu.VMEM((ACC_ROWS_PER_TILE, DW), jnp.float32),
            pltpu.VMEM((BLK, DW), jnp.float32),
            pltpu.VMEM((BLK,), jnp.int32),
            pltpu.VMEM((BLK,), jnp.int32),
            pltpu.SemaphoreType.DMA,
            pltpu.SemaphoreType.DMA,
        ],
    )
    def degree(colp_hbm, out_hbm, acc, zbuf, ones, idx_c0, idx_c1, semc0, semc1):
        c = lax.axis_index("c")
        s = lax.axis_index("s")
        t = s * NC + c
        tb = t * kbp * BLK

        def _cols(j):
            return colp_hbm.at[pl.ds(tb + j * BLK, BLK)]

        pltpu.async_copy(_cols(0), idx_c0, semc0)
        pltpu.async_copy(_cols(1), idx_c1, semc1)

        zeros16 = jnp.zeros((16,), jnp.float32)
        ones16 = jnp.ones((16,), jnp.float32)

        def _z(i, _):
            zbuf[i, :] = zeros16
            return 0

        def _o(i, _):
            ones[i, :] = ones16
            return 0

        lax.fori_loop(0, ACC_ROWS_PER_TILE, _z, 0)
        lax.fori_loop(0, BLK, _o, 0)

        pltpu.sync_copy(zbuf, acc.at[pl.ds(s * ACC_ROWS_PER_TILE, ACC_ROWS_PER_TILE)])
        plsc.subcore_barrier()

        def _pair(j2, _):
            j0 = 2 * j2
            pltpu.make_async_copy(_cols(0), idx_c0, semc0).wait()
            pltpu.sync_copy(ones, acc.at[idx_c0], add=True)
            pltpu.async_copy(_cols(j0 + 2), idx_c0, semc0)
            pltpu.make_async_copy(_cols(1), idx_c1, semc1).wait()
            pltpu.sync_copy(ones, acc.at[idx_c1], add=True)
            pltpu.async_copy(_cols(j0 + 3), idx_c1, semc1)
            return 0

        lax.fori_loop(0, kb // 2, _pair, 0)
        pltpu.make_async_copy(_cols(0), idx_c0, semc0).wait()
        pltpu.make_async_copy(_cols(1), idx_c1, semc1).wait()
        plsc.subcore_barrier()

        ob = s * OUT_ROWS
        pltpu.sync_copy(acc.at[pl.ds(ob, OUT_ROWS)], out_hbm.at[c, pl.ds(ob, OUT_ROWS)])

    return degree


def _dot(a, b):
    return jnp.dot(a, b, preferred_element_type=jnp.float32)


def _enc_body(x_r, wencT_r, b_r, w0_r, wi_r, we_r, p0_r, p1_r,
              x0_o, x0w0_o, dinv_o, xi_o, y_o, hwe_o):
    x = x_r[...]
    x0 = _dot(x, wencT_r[...]) + b_r[...]
    deg = p0_r[0, :, 0:1] + p1_r[0, :, 0:1] + 1.0
    dinvb = jnp.broadcast_to(lax.rsqrt(deg), x0.shape)
    xi = _dot(x0, wi_r[...])
    x0_o[...] = x0
    x0w0_o[...] = _dot(x0, w0_r[...])
    dinv_o[...] = dinvb
    xi_o[...] = xi
    y_o[...] = dinvb * xi
    hwe_o[...] = _dot(x0, we_r[...])


def _layer_body(h_r, xi_r, hwe_r, x0w0_r, dinv_r, p0_r, p1_r, wi_r, we_r,
                hn_o, xin_o, yn_o, hwen_o):
    dinvb = dinv_r[...]
    agg = dinvb * (p0_r[0] + p1_r[0]) + dinvb * dinvb * xi_r[...]
    hn = h_r[...] + jnp.maximum(agg - hwe_r[...] - x0w0_r[...], 0.0)
    hn = jnp.maximum(hn, 0.0)
    xin = _dot(hn, wi_r[...])
    hn_o[...] = hn
    xin_o[...] = xin
    yn_o[...] = dinvb * xin
    hwen_o[...] = _dot(hn, we_r[...])


def _final_body(h_r, xi_r, hwe_r, x0w0_r, dinv_r, p0_r, p1_r, wdecT_r, bd_r,
                out_o):
    dinvb = dinv_r[...]
    agg = dinvb * (p0_r[0] + p1_r[0]) + dinvb * dinvb * xi_r[...]
    hn = h_r[...] + jnp.maximum(agg - hwe_r[...] - x0w0_r[...], 0.0)
    hn = jnp.maximum(hn, 0.0)
    out_o[...] = _dot(hn, wdecT_r[...]) + bd_r[...]


def _row_spec():
    return pl.BlockSpec((R, D), lambda i: (i, 0))


def _pp_spec(core):
    return pl.BlockSpec((1, R, D), lambda i, core=core: (core, i, 0))


def _dp_spec(core):
    return pl.BlockSpec((1, R, DW), lambda i, core=core: (core, i, 0))


def _w_spec():
    return pl.BlockSpec((D, D), lambda i: (0, 0))


def _b_spec():
    return pl.BlockSpec((1, D), lambda i: (0, 0))


_F32 = functools.partial(jax.ShapeDtypeStruct, dtype=jnp.float32)

_encoder = pl.pallas_call(
    _enc_body,
    grid=(GRID,),
    in_specs=[_row_spec(), _w_spec(), _b_spec(), _w_spec(), _w_spec(), _w_spec(),
              _dp_spec(0), _dp_spec(1)],
    out_specs=[_row_spec()] * 6,
    out_shape=[_F32((N, D))] * 6,
)

_layer = pl.pallas_call(
    _layer_body,
    grid=(GRID,),
    in_specs=[_row_spec()] * 5 + [_pp_spec(0), _pp_spec(1), _w_spec(), _w_spec()],
    out_specs=[_row_spec()] * 4,
    out_shape=[_F32((N, D))] * 4,
)

_final = pl.pallas_call(
    _final_body,
    grid=(GRID,),
    in_specs=[_row_spec()] * 5 + [_pp_spec(0), _pp_spec(1), _w_spec(), _b_spec()],
    out_specs=_row_spec(),
    out_shape=_F32((N, D)),
)


def _symmetrize(W):
    return jnp.triu(W) + jnp.triu(W, 1).T


def kernel(x, edge_index, W_enc, b_enc, Wi, We, W0, W_dec, b_dec):
    row = edge_index[0].astype(jnp.int32)
    col = edge_index[1].astype(jnp.int32)
    E = row.shape[0]
    kb = -(-E // (NW * BLK))
    kb += kb % 2
    EP = NW * kb * BLK
    rowp = jnp.concatenate([row, jnp.zeros((EP - E,), jnp.int32)]).reshape(NW, kb, BLK)
    rowp = jnp.concatenate(
        [rowp, jnp.zeros((NW, 2, BLK), jnp.int32)], axis=1).reshape(-1)
    colp3 = jnp.concatenate([col, jnp.full((EP - E,), N, jnp.int32)]).reshape(NW, kb, BLK)
    colp = jnp.concatenate(
        [colp3, jnp.full((NW, 2, BLK), N, jnp.int32)], axis=1).reshape(-1)

    Wi_s = _symmetrize(Wi)
    We_s = _symmetrize(We)
    W0_s = _symmetrize(W0)

    degree = _make_degree(kb)
    propagate = _make_propagate(kb)

    dp = degree(colp)
    x0, x0w0, dinvb, xi, y, hwe = _encoder(
        x, W_enc.T, b_enc.reshape(1, D), W0_s, Wi_s, We_s, dp, dp)

    h = x0
    for _ in range(NUM_LAYERS - 1):
        pp = propagate(y, rowp, colp3)
        h, xi, y, hwe = _layer(h, xi, hwe, x0w0, dinvb, pp, pp, Wi_s, We_s)
    pp = propagate(y, rowp, colp3)
    return _final(h, xi, hwe, x0w0, dinvb, pp, pp, W_dec.T,
                  b_dec.reshape(1, D))


# speedup vs baseline: 1.4174x; 1.4174x over previous
import functools

import jax
import jax.numpy as jnp
from jax import lax
from jax.experimental import pallas as pl
from jax.experimental.pallas import tpu as pltpu
from jax.experimental.pallas import tpu_sc as plsc

N = 10000
D = 128
NUM_LAYERS = 4
NC = 2
NS = 16
NW = NC * NS
BLK = 128
ACC_ROWS_PER_TILE = 632
NPAD = NS * ACC_ROWS_PER_TILE
OUT_ROWS = ACC_ROWS_PER_TILE
DW = 16
R = 1000
GRID = N // R


def _make_propagate(q0, q1):
    mesh = plsc.VectorSubcoreMesh(core_axis_name="c", subcore_axis_name="s")
    slot = q0 + q1 + 4
    qmax = max(q0, q1)

    @functools.partial(
        pl.kernel,
        out_type=jax.ShapeDtypeStruct((NC, NPAD, D), jnp.float32),
        mesh=mesh,
        scratch_types=[
            pltpu.VMEM_SHARED((NPAD, D), jnp.float32),
            pltpu.VMEM((BLK, D), jnp.float32),
            pltpu.VMEM(((qmax + 2) * BLK,), jnp.int32),
            pltpu.VMEM((BLK,), jnp.int32),
            pltpu.VMEM((BLK,), jnp.int32),
            pltpu.SemaphoreType.DMA,
            pltpu.SemaphoreType.DMA,
            pltpu.SemaphoreType.DMA,
        ],
    )
    def propagate(y_hbm, rowp_hbm, colp_hbm, out_hbm, acc, buf0,
                  idx_r, idx_c0, idx_c1, sem0, semc0, semc1):
        c = lax.axis_index("c")
        s = lax.axis_index("s")
        kq2 = jnp.where(c == 0, q0 // 2, q1 // 2)
        tb = (s * slot + c * (q0 + 2)) * BLK

        def _cols(j):
            return colp_hbm.at[pl.ds(tb + j * BLK, BLK)]

        pltpu.async_copy(rowp_hbm.at[pl.ds(tb, (qmax + 2) * BLK)], idx_r, sem0)
        pltpu.async_copy(_cols(0), idx_c0, semc0)
        pltpu.async_copy(_cols(1), idx_c1, semc1)

        zeros16 = jnp.zeros((16,), jnp.float32)

        def _zrow(i, _):
            def _zcol(j, _):
                buf0[i, pl.ds(j * 16, 16)] = zeros16
                return 0
            return lax.fori_loop(0, D // 16, _zcol, 0)

        lax.fori_loop(0, BLK, _zrow, 0)

        zb = s * ACC_ROWS_PER_TILE
        for k in range(ACC_ROWS_PER_TILE // BLK):
            pltpu.sync_copy(buf0, acc.at[pl.ds(zb + k * BLK, BLK)])
        rem = ACC_ROWS_PER_TILE % BLK
        pltpu.sync_copy(buf0.at[pl.ds(0, rem)],
                        acc.at[pl.ds(zb + (ACC_ROWS_PER_TILE // BLK) * BLK, rem)])

        pltpu.make_async_copy(
            rowp_hbm.at[pl.ds(0, (qmax + 2) * BLK)], idx_r, sem0).wait()
        plsc.subcore_barrier()

        def _gidx(j):
            return idx_r.at[pl.ds(j * BLK, BLK)]

        def _pair(j2, _):
            j0 = 2 * j2
            pltpu.async_copy(y_hbm.at[_gidx(j0)], buf0, sem0).wait()
            pltpu.make_async_copy(_cols(0), idx_c0, semc0).wait()
            pltpu.sync_copy(buf0, acc.at[idx_c0], add=True)
            pltpu.async_copy(_cols(j0 + 2), idx_c0, semc0)
            pltpu.async_copy(y_hbm.at[_gidx(j0 + 1)], buf0, sem0).wait()
            pltpu.make_async_copy(_cols(1), idx_c1, semc1).wait()
            pltpu.sync_copy(buf0, acc.at[idx_c1], add=True)
            pltpu.async_copy(_cols(j0 + 3), idx_c1, semc1)
            return 0

        lax.fori_loop(0, kq2, _pair, 0)
        pltpu.make_async_copy(_cols(0), idx_c0, semc0).wait()
        pltpu.make_async_copy(_cols(1), idx_c1, semc1).wait()
        plsc.subcore_barrier()

        ob = s * OUT_ROWS
        pltpu.sync_copy(acc.at[pl.ds(ob, OUT_ROWS)], out_hbm.at[c, pl.ds(ob, OUT_ROWS)])

    return propagate


def _make_degree(q0, q1):
    mesh = plsc.VectorSubcoreMesh(core_axis_name="c", subcore_axis_name="s")
    slot = q0 + q1 + 4

    @functools.partial(
        pl.kernel,
        out_type=jax.ShapeDtypeStruct((NC, NPAD, DW), jnp.float32),
        mesh=mesh,
        scratch_types=[
            pltpu.VMEM_SHARED((NPAD, DW), jnp.float32),
            pltpu.VMEM((ACC_ROWS_PER_TILE, DW), jnp.float32),
            pltpu.VMEM((BLK, DW), jnp.float32),
            pltpu.VMEM((BLK,), jnp.int32),
            pltpu.VMEM((BLK,), jnp.int32),
            pltpu.SemaphoreType.DMA,
            pltpu.SemaphoreType.DMA,
        ],
    )
    def degree(colp_hbm, out_hbm, acc, zbuf, ones, idx_c0, idx_c1, semc0, semc1):
        c = lax.axis_index("c")
        s = lax.axis_index("s")
        kq2 = jnp.where(c == 0, q0 // 2, q1 // 2)
        tb = (s * slot + c * (q0 + 2)) * BLK

        def _cols(j):
            return colp_hbm.at[pl.ds(tb + j * BLK, BLK)]

        pltpu.async_copy(_cols(0), idx_c0, semc0)
        pltpu.async_copy(_cols(1), idx_c1, semc1)

        zeros16 = jnp.zeros((16,), jnp.float32)
        ones16 = jnp.ones((16,), jnp.float32)

        def _z(i, _):
            zbuf[i, :] = zeros16
            return 0

        def _o(i, _):
            ones[i, :] = ones16
            return 0

        lax.fori_loop(0, ACC_ROWS_PER_TILE, _z, 0)
        lax.fori_loop(0, BLK, _o, 0)

        pltpu.sync_copy(zbuf, acc.at[pl.ds(s * ACC_ROWS_PER_TILE, ACC_ROWS_PER_TILE)])
        plsc.subcore_barrier()

        def _pair(j2, _):
            j0 = 2 * j2
            pltpu.make_async_copy(_cols(0), idx_c0, semc0).wait()
            pltpu.sync_copy(ones, acc.at[idx_c0], add=True)
            pltpu.async_copy(_cols(j0 + 2), idx_c0, semc0)
            pltpu.make_async_copy(_cols(1), idx_c1, semc1).wait()
            pltpu.sync_copy(ones, acc.at[idx_c1], add=True)
            pltpu.async_copy(_cols(j0 + 3), idx_c1, semc1)
            return 0

        lax.fori_loop(0, kq2, _pair, 0)
        pltpu.make_async_copy(_cols(0), idx_c0, semc0).wait()
        pltpu.make_async_copy(_cols(1), idx_c1, semc1).wait()
        plsc.subcore_barrier()

        ob = s * OUT_ROWS
        pltpu.sync_copy(acc.at[pl.ds(ob, OUT_ROWS)], out_hbm.at[c, pl.ds(ob, OUT_ROWS)])

    return degree


def _dot(a, b):
    return jnp.dot(a, b, preferred_element_type=jnp.float32)


def _enc_body(x_r, wencT_r, b_r, w0_r, wi_r, we_r, p0_r, p1_r,
              x0_o, x0w0_o, dinv_o, xi_o, y_o, hwe_o):
    x = x_r[...]
    x0 = _dot(x, wencT_r[...]) + b_r[...]
    deg = p0_r[0, :, 0:1] + p1_r[0, :, 0:1] + 1.0
    dinvb = jnp.broadcast_to(lax.rsqrt(deg), x0.shape)
    xi = _dot(x0, wi_r[...])
    x0_o[...] = x0
    x0w0_o[...] = _dot(x0, w0_r[...])
    dinv_o[...] = dinvb
    xi_o[...] = xi
    y_o[...] = dinvb * xi
    hwe_o[...] = _dot(x0, we_r[...])


def _layer_body(h_r, xi_r, hwe_r, x0w0_r, dinv_r, p0_r, p1_r, wi_r, we_r,
                hn_o, xin_o, yn_o, hwen_o):
    dinvb = dinv_r[...]
    agg = dinvb * (p0_r[0] + p1_r[0]) + dinvb * dinvb * xi_r[...]
    hn = h_r[...] + jnp.maximum(agg - hwe_r[...] - x0w0_r[...], 0.0)
    hn = jnp.maximum(hn, 0.0)
    xin = _dot(hn, wi_r[...])
    hn_o[...] = hn
    xin_o[...] = xin
    yn_o[...] = dinvb * xin
    hwen_o[...] = _dot(hn, we_r[...])


def _final_body(h_r, xi_r, hwe_r, x0w0_r, dinv_r, p0_r, p1_r, wdecT_r, bd_r,
                out_o):
    dinvb = dinv_r[...]
    agg = dinvb * (p0_r[0] + p1_r[0]) + dinvb * dinvb * xi_r[...]
    hn = h_r[...] + jnp.maximum(agg - hwe_r[...] - x0w0_r[...], 0.0)
    hn = jnp.maximum(hn, 0.0)
    out_o[...] = _dot(hn, wdecT_r[...]) + bd_r[...]


def _row_spec():
    return pl.BlockSpec((R, D), lambda i: (i, 0))


def _pp_spec(core):
    return pl.BlockSpec((1, R, D), lambda i, core=core: (core, i, 0))


def _dp_spec(core):
    return pl.BlockSpec((1, R, DW), lambda i, core=core: (core, i, 0))


def _w_spec():
    return pl.BlockSpec((D, D), lambda i: (0, 0))


def _b_spec():
    return pl.BlockSpec((1, D), lambda i: (0, 0))


_F32 = functools.partial(jax.ShapeDtypeStruct, dtype=jnp.float32)

_encoder = pl.pallas_call(
    _enc_body,
    grid=(GRID,),
    in_specs=[_row_spec(), _w_spec(), _b_spec(), _w_spec(), _w_spec(), _w_spec(),
              _dp_spec(0), _dp_spec(1)],
    out_specs=[_row_spec()] * 6,
    out_shape=[_F32((N, D))] * 6,
)

_layer = pl.pallas_call(
    _layer_body,
    grid=(GRID,),
    in_specs=[_row_spec()] * 5 + [_pp_spec(0), _pp_spec(1), _w_spec(), _w_spec()],
    out_specs=[_row_spec()] * 4,
    out_shape=[_F32((N, D))] * 4,
)

_final = pl.pallas_call(
    _final_body,
    grid=(GRID,),
    in_specs=[_row_spec()] * 5 + [_pp_spec(0), _pp_spec(1), _w_spec(), _b_spec()],
    out_specs=_row_spec(),
    out_shape=_F32((N, D)),
)


def _symmetrize(W):
    return jnp.triu(W) + jnp.triu(W, 1).T


def kernel(x, edge_index, W_enc, b_enc, Wi, We, W0, W_dec, b_dec):
    row = edge_index[0].astype(jnp.int32)
    col = edge_index[1].astype(jnp.int32)
    E = row.shape[0]
    Q0, Q1 = 60, 98
    BT = NS * (Q0 + Q1)
    EP = BT * BLK

    def _quota_layout(v, fill):
        vp = jnp.concatenate([v, jnp.full((EP - E,), fill, jnp.int32)])
        b3 = vp.reshape(NS, Q0 + Q1, BLK)
        pad = jnp.full((NS, 2, BLK), fill, jnp.int32)
        return jnp.concatenate(
            [b3[:, :Q0], pad, b3[:, Q0:], pad], axis=1).reshape(-1)

    rowp = _quota_layout(row, 0)
    colp = _quota_layout(col, N)

    Wi_s = _symmetrize(Wi)
    We_s = _symmetrize(We)
    W0_s = _symmetrize(W0)

    degree = _make_degree(Q0, Q1)
    propagate = _make_propagate(Q0, Q1)

    dp = degree(colp)
    x0, x0w0, dinvb, xi, y, hwe = _encoder(
        x, W_enc.T, b_enc.reshape(1, D), W0_s, Wi_s, We_s, dp, dp)

    h = x0
    for _ in range(NUM_LAYERS - 1):
        pp = propagate(y, rowp, colp)
        h, xi, y, hwe = _layer(h, xi, hwe, x0w0, dinvb, pp, pp, Wi_s, We_s)
    pp = propagate(y, rowp, colp)
    return _final(h, xi, hwe, x0w0, dinvb, pp, pp, W_dec.T,
                  b_dec.reshape(1, D))
